# baseline (speedup 1.0000x reference)
import functools

import jax
import jax.numpy as jnp
from jax import lax
from jax.experimental import pallas as pl
from jax.experimental.pallas import tpu as pltpu
from jax.experimental.pallas import tpu_sc as plsc

L = 16
NC = 2
NS = 16
NW = NC * NS
G = 4


def _make_sc_kernel(BM, T, N, K):
    assert BM % (NW * G) == 0 and N % L == 0
    ROWS = BM // NW
    NGRP = ROWS // G
    HALF = K // 2
    mesh = plsc.VectorSubcoreMesh(core_axis_name="c", subcore_axis_name="s")

    @functools.partial(
        pl.kernel,
        mesh=mesh,
        out_type=jax.ShapeDtypeStruct((BM, N), jnp.float32),
        compiler_params=pltpu.CompilerParams(needs_layout_passes=False),
        scratch_types=[
            pltpu.VMEM((N,), jnp.float32),
            pltpu.VMEM((N,), jnp.int32),
            pltpu.VMEM((K, N), jnp.float32),
            pltpu.VMEM((L,), jnp.float32),
            pltpu.VMEM((G * T,), jnp.float32),
            pltpu.VMEM((G * T,), jnp.float32),
            pltpu.VMEM((G, N), jnp.float32),
            pltpu.VMEM((G, N), jnp.float32),
            pltpu.SemaphoreType.DMA,
            pltpu.SemaphoreType.DMA,
            pltpu.SemaphoreType.DMA,
            pltpu.SemaphoreType.DMA,
        ],
    )
    def sc_kernel(x_hbm, tq_hbm, mask_hbm, out_hbm,
                  tq_v, s_v, w_v, mask_v, xbuf0, xbuf1, obuf0, obuf1,
                  semx0, semx1, semo0, semo1):
        wid = lax.axis_index("s") * NC + lax.axis_index("c")
        base = wid * ROWS
        xbufs = (xbuf0, xbuf1)
        obufs = (obuf0, obuf1)
        semxs = (semx0, semx1)
        semos = (semo0, semo1)

        def grp_rows(g):
            return x_hbm.at[pl.ds((base + g * G) * T, G * T)]

        def grp_out(g):
            return out_hbm.at[pl.ds(base + g * G, G)]

        pltpu.async_copy(grp_rows(0), xbufs[0], semxs[0])
        pltpu.async_copy(grp_rows(1), xbufs[1], semxs[1])

        pltpu.sync_copy(tq_hbm, tq_v)
        pltpu.sync_copy(mask_hbm, mask_v.at[pl.ds(0, K)])
        mvec = mask_v[pl.ds(0, L)]
        mk = [mvec[k] for k in range(K)]

        @plsc.parallel_loop(0, N // L, unroll=2)
        def setup_body(i):
            sl = pl.ds(i * L, L)
            t = tq_v[sl]
            ti = t.astype(jnp.int32)
            tf = ti.astype(jnp.float32)
            s = jnp.where(tf < t, ti + 1, ti)
            s_v[sl] = s
            u = []
            for k in range(K):
                ik = s + (k - HALF)
                valid = (ik >= 0) & (ik < T)
                u.append(jnp.where(valid, mk[k], 0.0))
            norm = u[0]
            for k in range(1, K):
                norm = norm + u[k]
            inv = 1.0 / norm
            for k in range(K):
                w_v[k, sl] = u[k] * inv

        def pair_body(p, carry):
            for b in range(2):
                g = p * 2 + b
                xb, ob = xbufs[b], obufs[b]
                pltpu.make_async_copy(grp_rows(g), xb, semxs[b]).wait()
                @pl.when(g >= 2)
                def _():
                    pltpu.make_async_copy(ob, grp_out(g), semos[b]).wait()

                @plsc.parallel_loop(0, N // L, unroll=2)
                def q_body(i):
                    sl = pl.ds(i * L, L)
                    s = s_v[sl]
                    wk = [w_v[k, sl] if k != HALF else None
                          for k in range(K)]
                    wsum = None
                    for k in range(K):
                        if k != HALF:
                            wsum = wk[k] if wsum is None else wsum + wk[k]
                    wk[HALF] = 1.0 - wsum
                    idx = [jnp.clip(s + (k - HALF), 0, T - 1)
                           for k in range(K)]
                    for r in range(G):
                        acc = None
                        for k in range(K):
                            v = plsc.load_gather(xb, [idx[k] + (r * T)])
                            term = wk[k] * v
                            acc = term if acc is None else acc + term
                        ob[r, sl] = acc

                pltpu.async_copy(ob, grp_out(g), semos[b])
                @pl.when(g + 2 < NGRP)
                def _():
                    pltpu.async_copy(grp_rows(g + 2), xb, semxs[b])
            return carry

        lax.fori_loop(0, NGRP // 2, pair_body, 0)

        pltpu.make_async_copy(obufs[0], grp_out(NGRP - 2), semos[0]).wait()
        pltpu.make_async_copy(obufs[1], grp_out(NGRP - 1), semos[1]).wait()

    return sc_kernel


def kernel(surv_steps, time_bg, time_in, surv_conv_mask, single_time):
    B, M, T = surv_steps.shape
    N = time_in.shape[0]
    K = surv_conv_mask.shape[0]
    x1d = surv_steps.reshape(B * M * T)
    sc = _make_sc_kernel(B * M, T, N, K)
    out2d = sc(x1d, time_in, surv_conv_mask)
    return out2d.reshape(B, M, N)


# speedup vs baseline: 1.9644x; 1.9644x over previous
import functools

import jax
import jax.numpy as jnp
from jax import lax
from jax.experimental import pallas as pl
from jax.experimental.pallas import tpu as pltpu
from jax.experimental.pallas import tpu_sc as plsc

L = 16
NC = 2
NS = 16
NW = NC * NS
G = 4


def _make_sc_kernel(BM, T, N, K):
    assert BM % (NW * G) == 0 and N % L == 0
    ROWS = BM // NW
    NGRP = ROWS // G
    HALF = K // 2
    mesh = plsc.VectorSubcoreMesh(core_axis_name="c", subcore_axis_name="s")

    @functools.partial(
        pl.kernel,
        mesh=mesh,
        out_type=jax.ShapeDtypeStruct((BM, N), jnp.float32),
        compiler_params=pltpu.CompilerParams(needs_layout_passes=False),
        scratch_types=[
            pltpu.VMEM((N,), jnp.float32),
            pltpu.VMEM((N,), jnp.int32),
            pltpu.VMEM((K, N), jnp.float32),
            pltpu.VMEM((L,), jnp.float32),
            pltpu.VMEM((G, T), jnp.float32),
            pltpu.VMEM((G, T), jnp.float32),
            pltpu.VMEM((G, N), jnp.float32),
            pltpu.VMEM((G, N), jnp.float32),
            pltpu.SemaphoreType.DMA,
            pltpu.SemaphoreType.DMA,
            pltpu.SemaphoreType.DMA,
            pltpu.SemaphoreType.DMA,
        ],
    )
    def sc_kernel(x_hbm, tq_hbm, mask_hbm, out_hbm,
                  tq_v, s_v, w_v, mask_v, xbuf0, xbuf1, obuf0, obuf1,
                  semx0, semx1, semo0, semo1):
        wid = lax.axis_index("s") * NC + lax.axis_index("c")
        base = wid * ROWS
        xbufs = (xbuf0, xbuf1)
        obufs = (obuf0, obuf1)
        semxs = (semx0, semx1)
        semos = (semo0, semo1)

        def grp_rows(g):
            return x_hbm.at[pl.ds(base + g * G, G)]

        def grp_out(g):
            return out_hbm.at[pl.ds(base + g * G, G)]

        pltpu.async_copy(grp_rows(0), xbufs[0], semxs[0])
        pltpu.async_copy(grp_rows(1), xbufs[1], semxs[1])

        pltpu.sync_copy(tq_hbm, tq_v)
        pltpu.sync_copy(mask_hbm, mask_v.at[pl.ds(0, K)])
        mvec = mask_v[pl.ds(0, L)]
        mk = [mvec[k] for k in range(K)]

        @plsc.parallel_loop(0, N // L, unroll=2)
        def setup_body(i):
            sl = pl.ds(i * L, L)
            t = tq_v[sl]
            ti = t.astype(jnp.int32)
            tf = ti.astype(jnp.float32)
            s = jnp.where(tf < t, ti + 1, ti)
            s_v[sl] = s
            u = []
            for k in range(K):
                ik = s + (k - HALF)
                valid = (ik >= 0) & (ik < T)
                u.append(jnp.where(valid, mk[k], 0.0))
            norm = u[0]
            for k in range(1, K):
                norm = norm + u[k]
            inv = 1.0 / norm
            for k in range(K):
                w_v[k, sl] = u[k] * inv

        def pair_body(p, carry):
            for b in range(2):
                g = p * 2 + b
                xb, ob = xbufs[b], obufs[b]
                pltpu.make_async_copy(grp_rows(g), xb, semxs[b]).wait()
                @pl.when(g >= 2)
                def _():
                    pltpu.make_async_copy(ob, grp_out(g), semos[b]).wait()

                @plsc.parallel_loop(0, N // L, unroll=2)
                def q_body(i):
                    sl = pl.ds(i * L, L)
                    s = s_v[sl]
                    wk = [w_v[k, sl] if k != HALF else None
                          for k in range(K)]
                    wsum = None
                    for k in range(K):
                        if k != HALF:
                            wsum = wk[k] if wsum is None else wsum + wk[k]
                    wk[HALF] = 1.0 - wsum
                    idx = [jnp.clip(s + (k - HALF), 0, T - 1)
                           for k in range(K)]
                    for r in range(G):
                        rvec = jnp.full((L,), r, jnp.int32)
                        acc = None
                        for k in range(K):
                            v = plsc.load_gather(xb, [rvec, idx[k]])
                            term = wk[k] * v
                            acc = term if acc is None else acc + term
                        ob[r, sl] = acc

                pltpu.async_copy(ob, grp_out(g), semos[b])
                @pl.when(g + 2 < NGRP)
                def _():
                    pltpu.async_copy(grp_rows(g + 2), xb, semxs[b])
            return carry

        lax.fori_loop(0, NGRP // 2, pair_body, 0)

        pltpu.make_async_copy(obufs[0], grp_out(NGRP - 2), semos[0]).wait()
        pltpu.make_async_copy(obufs[1], grp_out(NGRP - 1), semos[1]).wait()

    return sc_kernel


def kernel(surv_steps, time_bg, time_in, surv_conv_mask, single_time):
    B, M, T = surv_steps.shape
    N = time_in.shape[0]
    K = surv_conv_mask.shape[0]
    x2d = surv_steps.reshape(B * M, T)
    sc = _make_sc_kernel(B * M, T, N, K)
    out2d = sc(x2d, time_in, surv_conv_mask)
    return out2d.reshape(B, M, N)


# speedup vs baseline: 1.9684x; 1.0020x over previous
import functools

import jax
import jax.numpy as jnp
from jax import lax
from jax.experimental import pallas as pl
from jax.experimental.pallas import tpu as pltpu
from jax.experimental.pallas import tpu_sc as plsc

L = 16
NC = 2
NS = 16
NW = NC * NS
G = 4


def _make_sc_kernel(BM, T, N, K):
    assert BM % (NW * G) == 0 and N % L == 0
    ROWS = BM // NW
    NGRP = ROWS // G
    HALF = K // 2
    mesh = plsc.VectorSubcoreMesh(core_axis_name="c", subcore_axis_name="s")

    @functools.partial(
        pl.kernel,
        mesh=mesh,
        out_type=jax.ShapeDtypeStruct((BM, N), jnp.float32),
        compiler_params=pltpu.CompilerParams(needs_layout_passes=False),
        scratch_types=[
            pltpu.VMEM((N,), jnp.float32),
            pltpu.VMEM((N,), jnp.int32),
            pltpu.VMEM((K, N), jnp.float32),
            pltpu.VMEM((L,), jnp.float32),
            pltpu.VMEM((G, T), jnp.float32),
            pltpu.VMEM((G, T), jnp.float32),
            pltpu.VMEM((G, N), jnp.float32),
            pltpu.VMEM((G, N), jnp.float32),
            pltpu.SemaphoreType.DMA,
            pltpu.SemaphoreType.DMA,
            pltpu.SemaphoreType.DMA,
            pltpu.SemaphoreType.DMA,
        ],
    )
    def sc_kernel(x_hbm, tq_hbm, mask_hbm, out_hbm,
                  tq_v, s_v, w_v, mask_v, xbuf0, xbuf1, obuf0, obuf1,
                  semx0, semx1, semo0, semo1):
        wid = lax.axis_index("s") * NC + lax.axis_index("c")
        base = wid * ROWS
        xbufs = (xbuf0, xbuf1)
        obufs = (obuf0, obuf1)
        semxs = (semx0, semx1)
        semos = (semo0, semo1)

        def grp_rows(g):
            return x_hbm.at[pl.ds(base + g * G, G)]

        def grp_out(g):
            return out_hbm.at[pl.ds(base + g * G, G)]

        pltpu.async_copy(grp_rows(0), xbufs[0], semxs[0])
        pltpu.async_copy(grp_rows(1), xbufs[1], semxs[1])

        pltpu.sync_copy(tq_hbm, tq_v)
        pltpu.sync_copy(mask_hbm, mask_v.at[pl.ds(0, K)])
        mvec = mask_v[pl.ds(0, L)]
        mk = [mvec[k] for k in range(K)]

        @plsc.parallel_loop(0, N // L, unroll=2)
        def setup_body(i):
            sl = pl.ds(i * L, L)
            t = tq_v[sl]
            ti = t.astype(jnp.int32)
            tf = ti.astype(jnp.float32)
            s = jnp.where(tf < t, ti + 1, ti)
            s_v[sl] = s
            u = []
            for k in range(K):
                ik = s + (k - HALF)
                valid = (ik >= 0) & (ik < T)
                u.append(jnp.where(valid, mk[k], 0.0))
            norm = u[0]
            for k in range(1, K):
                norm = norm + u[k]
            inv = 1.0 / norm
            for k in range(K):
                w_v[k, sl] = u[k] * inv

        def pair_body(p, carry):
            for b in range(2):
                g = p * 2 + b
                xb, ob = xbufs[b], obufs[b]
                pltpu.make_async_copy(grp_rows(g), xb, semxs[b]).wait()
                @pl.when(g >= 2)
                def _():
                    pltpu.make_async_copy(ob, grp_out(g), semos[b]).wait()

                @plsc.parallel_loop(0, N // L, unroll=2)
                def q_body(i):
                    sl = pl.ds(i * L, L)
                    s = s_v[sl]
                    wk = [w_v[k, sl] if k != HALF else None
                          for k in range(K)]
                    wsum = None
                    for k in range(K):
                        if k != HALF:
                            wsum = wk[k] if wsum is None else wsum + wk[k]
                    wk[HALF] = 1.0 - wsum
                    idx = [jnp.clip(s + (k - HALF), 0, T - 1)
                           for k in range(K)]
                    for r in range(G):
                        rvec = jnp.full((L,), r, jnp.int32)
                        acc = None
                        for k in range(K):
                            v = plsc.load_gather(xb, [rvec, idx[k]])
                            term = wk[k] * v
                            acc = term if acc is None else acc + term
                        ob[r, sl] = acc

                @pl.when(g + 2 < NGRP)
                def _():
                    pltpu.async_copy(grp_rows(g + 2), xb, semxs[b])
                pltpu.async_copy(ob, grp_out(g), semos[b])
            return carry

        lax.fori_loop(0, NGRP // 2, pair_body, 0)

        pltpu.make_async_copy(obufs[0], grp_out(NGRP - 2), semos[0]).wait()
        pltpu.make_async_copy(obufs[1], grp_out(NGRP - 1), semos[1]).wait()

    return sc_kernel


def kernel(surv_steps, time_bg, time_in, surv_conv_mask, single_time):
    B, M, T = surv_steps.shape
    N = time_in.shape[0]
    K = surv_conv_mask.shape[0]
    x2d = surv_steps.reshape(B * M, T)
    sc = _make_sc_kernel(B * M, T, N, K)
    out2d = sc(x2d, time_in, surv_conv_mask)
    return out2d.reshape(B, M, N)


# speedup vs baseline: 2.1063x; 1.0701x over previous
import functools

import jax
import jax.numpy as jnp
from jax import lax
from jax.experimental import pallas as pl
from jax.experimental.pallas import tpu as pltpu
from jax.experimental.pallas import tpu_sc as plsc

L = 16
NC = 2
NS = 16
NW = NC * NS
G = 4


def _make_sc_kernel(BM, T, N, K):
    assert BM % (NW * G) == 0 and N % L == 0
    ROWS = BM // NW
    NGRP = ROWS // G
    HALF = K // 2
    mesh = plsc.VectorSubcoreMesh(core_axis_name="c", subcore_axis_name="s")

    @functools.partial(
        pl.kernel,
        mesh=mesh,
        out_type=jax.ShapeDtypeStruct((BM, N), jnp.float32),
        compiler_params=pltpu.CompilerParams(needs_layout_passes=False),
        scratch_types=[
            pltpu.VMEM((N,), jnp.float32),
            pltpu.VMEM((N,), jnp.int32),
            pltpu.VMEM((K, N), jnp.float32),
            pltpu.VMEM((L,), jnp.float32),
            pltpu.VMEM((G, T), jnp.float32),
            pltpu.VMEM((G, T), jnp.float32),
            pltpu.VMEM((G, T), jnp.float32),
            pltpu.VMEM((G, N), jnp.float32),
            pltpu.VMEM((G, N), jnp.float32),
            pltpu.VMEM((G, N), jnp.float32),
            pltpu.SemaphoreType.DMA,
            pltpu.SemaphoreType.DMA,
            pltpu.SemaphoreType.DMA,
            pltpu.SemaphoreType.DMA,
            pltpu.SemaphoreType.DMA,
            pltpu.SemaphoreType.DMA,
        ],
    )
    def sc_kernel(x_hbm, tq_hbm, mask_hbm, out_hbm,
                  tq_v, s_v, w_v, mask_v, xbuf0, xbuf1, xbuf2,
                  obuf0, obuf1, obuf2, semx0, semx1, semx2,
                  semo0, semo1, semo2):
        wid = lax.axis_index("s") * NC + lax.axis_index("c")
        base = wid * ROWS
        xbufs = (xbuf0, xbuf1, xbuf2)
        obufs = (obuf0, obuf1, obuf2)
        semxs = (semx0, semx1, semx2)
        semos = (semo0, semo1, semo2)

        def grp_rows(g):
            return x_hbm.at[pl.ds(base + g * G, G)]

        def grp_out(g):
            return out_hbm.at[pl.ds(base + g * G, G)]

        pltpu.async_copy(grp_rows(0), xbufs[0], semxs[0])
        pltpu.async_copy(grp_rows(1), xbufs[1], semxs[1])

        pltpu.sync_copy(tq_hbm, tq_v)
        pltpu.sync_copy(mask_hbm, mask_v.at[pl.ds(0, K)])
        mvec = mask_v[pl.ds(0, L)]
        mk = [mvec[k] for k in range(K)]

        @plsc.parallel_loop(0, N // L, unroll=2)
        def setup_body(i):
            sl = pl.ds(i * L, L)
            t = tq_v[sl]
            ti = t.astype(jnp.int32)
            tf = ti.astype(jnp.float32)
            s = jnp.where(tf < t, ti + 1, ti)
            s_v[sl] = s
            u = []
            for k in range(K):
                ik = s + (k - HALF)
                valid = (ik >= 0) & (ik < T)
                u.append(jnp.where(valid, mk[k], 0.0))
            norm = u[0]
            for k in range(1, K):
                norm = norm + u[k]
            inv = 1.0 / norm
            for k in range(K):
                w_v[k, sl] = u[k] * inv

        def grp_step(g, b):
            xb, ob = xbufs[b], obufs[b]
            pltpu.make_async_copy(grp_rows(g), xb, semxs[b]).wait()
            @pl.when(g + 2 < NGRP)
            def _():
                pltpu.async_copy(grp_rows(g + 2), xbufs[(b + 2) % 3],
                                 semxs[(b + 2) % 3])
            @pl.when(g >= 3)
            def _():
                pltpu.make_async_copy(ob, grp_out(g), semos[b]).wait()

            @plsc.parallel_loop(0, N // L, unroll=2)
            def q_body(i):
                sl = pl.ds(i * L, L)
                s = s_v[sl]
                wk = [w_v[k, sl] if k != HALF else None
                      for k in range(K)]
                wsum = None
                for k in range(K):
                    if k != HALF:
                        wsum = wk[k] if wsum is None else wsum + wk[k]
                wk[HALF] = 1.0 - wsum
                idx = [jnp.clip(s + (k - HALF), 0, T - 1)
                       for k in range(K)]
                for r in range(G):
                    rvec = jnp.full((L,), r, jnp.int32)
                    acc = None
                    for k in range(K):
                        v = plsc.load_gather(xb, [rvec, idx[k]])
                        term = wk[k] * v
                        acc = term if acc is None else acc + term
                    ob[r, sl] = acc

            pltpu.async_copy(ob, grp_out(g), semos[b])

        def trio_body(p, carry):
            for j in range(3):
                grp_step(p * 3 + j, j)
            return carry

        lax.fori_loop(0, NGRP // 3, trio_body, 0)
        for g in range((NGRP // 3) * 3, NGRP):
            grp_step(g, g % 3)

        for g in range(NGRP - 3, NGRP):
            pltpu.make_async_copy(obufs[g % 3], grp_out(g),
                                  semos[g % 3]).wait()

    return sc_kernel


def kernel(surv_steps, time_bg, time_in, surv_conv_mask, single_time):
    B, M, T = surv_steps.shape
    N = time_in.shape[0]
    K = surv_conv_mask.shape[0]
    x2d = surv_steps.reshape(B * M, T)
    sc = _make_sc_kernel(B * M, T, N, K)
    out2d = sc(x2d, time_in, surv_conv_mask)
    return out2d.reshape(B, M, N)


# speedup vs baseline: 2.1147x; 1.0040x over previous
import functools

import jax
import jax.numpy as jnp
from jax import lax
from jax.experimental import pallas as pl
from jax.experimental.pallas import tpu as pltpu
from jax.experimental.pallas import tpu_sc as plsc

L = 16
NC = 2
NS = 16
NW = NC * NS
G = 4


def _make_sc_kernel(BM, T, N, K):
    assert BM % (NW * G) == 0 and N % L == 0
    ROWS = BM // NW
    NGRP = ROWS // G
    HALF = K // 2
    mesh = plsc.VectorSubcoreMesh(core_axis_name="c", subcore_axis_name="s")

    @functools.partial(
        pl.kernel,
        mesh=mesh,
        out_type=jax.ShapeDtypeStruct((BM, N), jnp.float32),
        compiler_params=pltpu.CompilerParams(needs_layout_passes=False),
        scratch_types=[
            pltpu.VMEM((N,), jnp.float32),
            pltpu.VMEM((N,), jnp.int32),
            pltpu.VMEM((K, N), jnp.float32),
            pltpu.VMEM((L,), jnp.float32),
            pltpu.VMEM((G, T), jnp.float32),
            pltpu.VMEM((G, T), jnp.float32),
            pltpu.VMEM((G, T), jnp.float32),
            pltpu.VMEM((G, N), jnp.float32),
            pltpu.VMEM((G, N), jnp.float32),
            pltpu.VMEM((G, N), jnp.float32),
            pltpu.SemaphoreType.DMA,
            pltpu.SemaphoreType.DMA,
            pltpu.SemaphoreType.DMA,
            pltpu.SemaphoreType.DMA,
            pltpu.SemaphoreType.DMA,
            pltpu.SemaphoreType.DMA,
        ],
    )
    def sc_kernel(x_hbm, tq_hbm, mask_hbm, out_hbm,
                  tq_v, s_v, w_v, mask_v, xbuf0, xbuf1, xbuf2,
                  obuf0, obuf1, obuf2, semx0, semx1, semx2,
                  semo0, semo1, semo2):
        wid = lax.axis_index("s") * NC + lax.axis_index("c")
        base = wid * ROWS
        xbufs = (xbuf0, xbuf1, xbuf2)
        obufs = (obuf0, obuf1, obuf2)
        semxs = (semx0, semx1, semx2)
        semos = (semo0, semo1, semo2)

        def grp_rows(g):
            return x_hbm.at[pl.ds(base + g * G, G)]

        def grp_out(g):
            return out_hbm.at[pl.ds(base + g * G, G)]

        pltpu.async_copy(grp_rows(0), xbufs[0], semxs[0])
        pltpu.async_copy(grp_rows(1), xbufs[1], semxs[1])

        pltpu.sync_copy(tq_hbm, tq_v)
        pltpu.sync_copy(mask_hbm, mask_v.at[pl.ds(0, K)])
        mvec = mask_v[pl.ds(0, L)]
        mk = [mvec[k] for k in range(K)]

        @plsc.parallel_loop(0, N // L, unroll=2)
        def setup_body(i):
            sl = pl.ds(i * L, L)
            t = tq_v[sl]
            ti = t.astype(jnp.int32)
            tf = ti.astype(jnp.float32)
            s = jnp.where(tf < t, ti + 1, ti)
            s_v[sl] = s
            u = []
            for k in range(K):
                ik = s + (k - HALF)
                valid = (ik >= 0) & (ik < T)
                u.append(jnp.where(valid, mk[k], 0.0))
            norm = u[0]
            for k in range(1, K):
                norm = norm + u[k]
            inv = 1.0 / norm
            for k in range(K):
                w_v[k, sl] = u[k] * inv

        def grp_step(g, b):
            xb, ob = xbufs[b], obufs[b]
            @pl.when(g + 2 < NGRP)
            def _():
                pltpu.async_copy(grp_rows(g + 2), xbufs[(b + 2) % 3],
                                 semxs[(b + 2) % 3])
            pltpu.make_async_copy(grp_rows(g), xb, semxs[b]).wait()
            @pl.when(g >= 3)
            def _():
                pltpu.make_async_copy(ob, grp_out(g), semos[b]).wait()

            @plsc.parallel_loop(0, N // L, unroll=2)
            def q_body(i):
                sl = pl.ds(i * L, L)
                s = s_v[sl]
                wk = [w_v[k, sl] if k != HALF else None
                      for k in range(K)]
                wsum = None
                for k in range(K):
                    if k != HALF:
                        wsum = wk[k] if wsum is None else wsum + wk[k]
                wk[HALF] = 1.0 - wsum
                idx = [jnp.clip(s + (k - HALF), 0, T - 1)
                       for k in range(K)]
                for r in range(G):
                    rvec = jnp.full((L,), r, jnp.int32)
                    acc = None
                    for k in range(K):
                        v = plsc.load_gather(xb, [rvec, idx[k]])
                        term = wk[k] * v
                        acc = term if acc is None else acc + term
                    ob[r, sl] = acc

            pltpu.async_copy(ob, grp_out(g), semos[b])

        def trio_body(p, carry):
            for j in range(3):
                grp_step(p * 3 + j, j)
            return carry

        lax.fori_loop(0, NGRP // 3, trio_body, 0)
        for g in range((NGRP // 3) * 3, NGRP):
            grp_step(g, g % 3)

        for g in range(NGRP - 3, NGRP):
            pltpu.make_async_copy(obufs[g % 3], grp_out(g),
                                  semos[g % 3]).wait()

    return sc_kernel


def kernel(surv_steps, time_bg, time_in, surv_conv_mask, single_time):
    B, M, T = surv_steps.shape
    N = time_in.shape[0]
    K = surv_conv_mask.shape[0]
    x2d = surv_steps.reshape(B * M, T)
    sc = _make_sc_kernel(B * M, T, N, K)
    out2d = sc(x2d, time_in, surv_conv_mask)
    return out2d.reshape(B, M, N)
